# Initial kernel scaffold; baseline (speedup 1.0000x reference)
import jax
import jax.numpy as jnp
from jax.experimental import pallas as pl


def kernel(x, edge_index, W, b, gamma, beta):
    raise NotImplementedError("write your pallas kernel here")


# speedup vs baseline: 12.3802x; 12.3802x over previous
import functools

import jax
import jax.numpy as jnp
from jax import lax
from jax.experimental import pallas as pl
from jax.experimental.pallas import tpu as pltpu
from jax.experimental.pallas import tpu_sc as plsc

N = 10000
D = 128
E = 320000
EPS = 1e-5

NC = 2
NS = 16
NW = NC * NS
K = 128
CHUNKS = 80
NBUF = 4
EP = K * CHUNKS
E_PAD = EP * NW
N_PAD = 10240
SL_DEG = N_PAD // NS
ROWS_W = N // NS
ZR = 16

_mesh = plsc.VectorSubcoreMesh(core_axis_name="c", subcore_axis_name="s")


DW = 128


@functools.partial(
    pl.kernel,
    out_type=jax.ShapeDtypeStruct((NC, N_PAD, DW), jnp.float32),
    mesh=_mesh,
    scratch_types=[
        pltpu.VMEM((CHUNKS, K), jnp.int32),
        pltpu.VMEM((K, DW), jnp.float32),
        pltpu.VMEM((64, DW), jnp.float32),
        pltpu.VMEM_SHARED((N_PAD, DW), jnp.float32),
    ],
)
def _deg_kernel(col_hbm, deg_out, cidx, onesbuf, zbuf, acc):
    cid = lax.axis_index("c")
    sid = lax.axis_index("s")
    wid = cid * NS + sid
    zero16 = jnp.zeros((16,), jnp.float32)
    ones16 = jnp.ones((16,), jnp.float32)
    for r in range(K):
        for j in range(DW // 16):
            onesbuf[r, pl.ds(j * 16, 16)] = ones16
    for r in range(64):
        for j in range(DW // 16):
            zbuf[r, pl.ds(j * 16, 16)] = zero16

    pltpu.sync_copy(col_hbm.at[wid], cidx)

    def zibody(i, _):
        pltpu.sync_copy(zbuf, acc.at[pl.ds(sid * SL_DEG + i * 64, 64)])
        return 0

    lax.fori_loop(0, SL_DEG // 64, zibody, 0)
    plsc.subcore_barrier()

    def ebody(g, _):
        pltpu.sync_copy(onesbuf, acc.at[cidx.at[g]], add=True)
        return 0

    lax.fori_loop(0, CHUNKS, ebody, 0)
    plsc.subcore_barrier()

    pltpu.sync_copy(
        acc.at[pl.ds(sid * SL_DEG, SL_DEG)],
        deg_out.at[cid, pl.ds(sid * SL_DEG, SL_DEG)],
    )


@functools.partial(
    pl.kernel,
    out_type=jax.ShapeDtypeStruct((NC, N_PAD, D), jnp.float32),
    mesh=_mesh,
    scratch_types=[
        pltpu.VMEM((CHUNKS, K), jnp.int32),
        pltpu.VMEM((CHUNKS, K), jnp.int32),
        pltpu.VMEM((K, D), jnp.float32),
        pltpu.VMEM((ZR, D), jnp.float32),
        pltpu.VMEM_SHARED((N_PAD, D), jnp.float32),
        pltpu.SemaphoreType.DMA,
    ],
)
def _scatter_kernel(hs_hbm, row_hbm, col_hbm, out_hbm, ridx, cidx, gbuf, zbuf, acc, sem):
    cid = lax.axis_index("c")
    sid = lax.axis_index("s")
    wid = cid * NS + sid
    zero16 = jnp.zeros((16,), jnp.float32)
    for r in range(ZR):
        for j in range(D // 16):
            zbuf[r, pl.ds(j * 16, 16)] = zero16

    pltpu.sync_copy(row_hbm.at[wid], ridx)
    pltpu.sync_copy(col_hbm.at[wid], cidx)

    def zibody(i, _):
        pltpu.sync_copy(zbuf, acc.at[pl.ds(sid * (N_PAD // NS) + i * ZR, ZR)])
        return 0

    lax.fori_loop(0, (N_PAD // NS) // ZR, zibody, 0)
    plsc.subcore_barrier()

    def ebody(g, _):
        pltpu.async_copy(hs_hbm.at[ridx.at[g]], gbuf, sem).wait()
        pltpu.sync_copy(gbuf, acc.at[cidx.at[g]], add=True)
        return 0

    lax.fori_loop(0, CHUNKS, ebody, 0)
    plsc.subcore_barrier()

    pltpu.sync_copy(
        acc.at[pl.ds(sid * (N_PAD // NS), N_PAD // NS)],
        out_hbm.at[cid, pl.ds(sid * (N_PAD // NS), N_PAD // NS)],
    )


BR = 400


def _mm_body(x_ref, w_ref, dega_ref, degb_ref, hs_ref, dinv_ref):
    deg = dega_ref[...] + degb_ref[...] + 1.0
    dinv = lax.rsqrt(deg)
    h = jnp.dot(x_ref[...], w_ref[...], preferred_element_type=jnp.float32)
    hs_ref[...] = h * dinv
    dinv_ref[...] = dinv


def _fin_body(acc_ref, hs_ref, dinv_ref, b_ref, gamma_ref, beta_ref, out_ref):
    pre = (acc_ref[0] + acc_ref[1] + hs_ref[...]) * dinv_ref[...] + b_ref[...]
    mean = jnp.mean(pre, axis=0, keepdims=True)
    var = jnp.mean(pre * pre, axis=0, keepdims=True) - mean * mean
    xhat = (pre - mean) * lax.rsqrt(var + EPS)
    o = gamma_ref[...] * xhat + beta_ref[...]
    out_ref[...] = o * (1.0 / (1.0 + jnp.exp(-o)))


@jax.jit
def _impl(x, edge_index, W, b, gamma, beta):
    row = edge_index[0]
    col = edge_index[1]
    pad = E_PAD - E
    row_p = jnp.concatenate([row, jnp.zeros((pad,), jnp.int32)]).reshape(NW, CHUNKS, K)
    col_p = jnp.concatenate([col, jnp.full((pad,), N, jnp.int32)]).reshape(NW, CHUNKS, K)

    deg2 = _deg_kernel(col_p)
    dega = deg2[0, :N, 0:1]
    degb = deg2[1, :N, 0:1]

    hs, dinv = pl.pallas_call(
        _mm_body,
        grid=(N // BR,),
        in_specs=[
            pl.BlockSpec((BR, D), lambda i: (i, 0)),
            pl.BlockSpec((D, D), lambda i: (0, 0)),
            pl.BlockSpec((BR, 1), lambda i: (i, 0)),
            pl.BlockSpec((BR, 1), lambda i: (i, 0)),
        ],
        out_specs=[
            pl.BlockSpec((BR, D), lambda i: (i, 0)),
            pl.BlockSpec((BR, 1), lambda i: (i, 0)),
        ],
        out_shape=[
            jax.ShapeDtypeStruct((N, D), jnp.float32),
            jax.ShapeDtypeStruct((N, 1), jnp.float32),
        ],
    )(x, W, dega, degb)

    acc2 = _scatter_kernel(hs, row_p, col_p)[:, :N, :]

    out = pl.pallas_call(
        _fin_body,
        out_shape=jax.ShapeDtypeStruct((N, D), jnp.float32),
    )(acc2, hs, dinv, b.reshape(1, D), gamma.reshape(1, D), beta.reshape(1, D))
    return out


def kernel(x, edge_index, W, b, gamma, beta):
    return _impl(x, edge_index, W, b, gamma, beta)


# speedup vs baseline: 15.1032x; 1.2199x over previous
import functools

import jax
import jax.numpy as jnp
from jax import lax
from jax.experimental import pallas as pl
from jax.experimental.pallas import tpu as pltpu
from jax.experimental.pallas import tpu_sc as plsc

N = 10000
D = 128
E = 320000
EPS = 1e-5

NC = 2
NS = 16
NW = NC * NS
K = 96
CHUNKS = 106
NBUF = 2
EP = K * CHUNKS
E_PAD = EP * NW
N_PAD = 10240
SL_DEG = N_PAD // NS
N_ACC = 10112
SL_ACC = N_ACC // NS
ZR = 16

_mesh = plsc.VectorSubcoreMesh(core_axis_name="c", subcore_axis_name="s")


DW = 128


@functools.partial(
    pl.kernel,
    out_type=jax.ShapeDtypeStruct((NC, N_PAD, DW), jnp.float32),
    mesh=_mesh,
    scratch_types=[
        pltpu.VMEM((CHUNKS, K), jnp.int32),
        pltpu.VMEM((K, DW), jnp.float32),
        pltpu.VMEM((64, DW), jnp.float32),
        pltpu.VMEM_SHARED((N_PAD, DW), jnp.float32),
    ],
)
def _deg_kernel(col_hbm, deg_out, cidx, onesbuf, zbuf, acc):
    cid = lax.axis_index("c")
    sid = lax.axis_index("s")
    wid = cid * NS + sid
    zero16 = jnp.zeros((16,), jnp.float32)
    ones16 = jnp.ones((16,), jnp.float32)
    for r in range(K):
        for j in range(DW // 16):
            onesbuf[r, pl.ds(j * 16, 16)] = ones16
    for r in range(64):
        for j in range(DW // 16):
            zbuf[r, pl.ds(j * 16, 16)] = zero16

    pltpu.sync_copy(col_hbm.at[wid], cidx)

    def zibody(i, _):
        pltpu.sync_copy(zbuf, acc.at[pl.ds(sid * SL_DEG + i * 64, 64)])
        return 0

    lax.fori_loop(0, SL_DEG // 64, zibody, 0)
    plsc.subcore_barrier()

    def ebody(g, _):
        pltpu.sync_copy(onesbuf, acc.at[cidx.at[g]], add=True)
        return 0

    lax.fori_loop(0, CHUNKS, ebody, 0)
    plsc.subcore_barrier()

    pltpu.sync_copy(
        acc.at[pl.ds(sid * SL_DEG, SL_DEG)],
        deg_out.at[cid, pl.ds(sid * SL_DEG, SL_DEG)],
    )


@functools.partial(
    pl.kernel,
    out_type=jax.ShapeDtypeStruct((NC, N_ACC, D), jnp.float32),
    mesh=_mesh,
    scratch_types=[
        pltpu.VMEM((EP,), jnp.int32),
        pltpu.VMEM((CHUNKS, K), jnp.int32),
        [pltpu.VMEM((K, D), jnp.float32) for _ in range(NBUF)],
        pltpu.VMEM_SHARED((N_ACC, D), jnp.float32),
        [pltpu.SemaphoreType.DMA for _ in range(NBUF)],
    ],
)
def _scatter_kernel(hs_hbm, row_hbm, col_hbm, out_hbm, ridx, cidx, gbufs, acc, gsems):
    cid = lax.axis_index("c")
    sid = lax.axis_index("s")
    wid = cid * NS + sid
    zero16 = jnp.zeros((16,), jnp.float32)
    for r in range(K):
        for j in range(D // 16):
            gbufs[0][r, pl.ds(j * 16, 16)] = zero16

    pltpu.sync_copy(row_hbm.at[wid], ridx)
    pltpu.sync_copy(col_hbm.at[wid], cidx)

    zbase = sid * SL_ACC
    for i in range(SL_ACC // K):
        pltpu.sync_copy(gbufs[0], acc.at[pl.ds(zbase + i * K, K)])
    zrem = SL_ACC % K
    if zrem:
        pltpu.sync_copy(
            gbufs[0].at[pl.ds(0, zrem)],
            acc.at[pl.ds(zbase + (SL_ACC // K) * K, zrem)],
        )
    plsc.subcore_barrier()

    pltpu.async_copy(hs_hbm.at[ridx.at[pl.ds(0, K)]], gbufs[0], gsems[0])

    def ebody(o, _):
        for b in range(NBUF):
            g = o * NBUF + b
            nb = 1 - b
            pltpu.async_copy(
                hs_hbm.at[ridx.at[pl.ds((g + 1) * K, K)]], gbufs[nb], gsems[nb]
            )
            pltpu.make_async_copy(
                hs_hbm.at[ridx.at[pl.ds(g * K, K)]], gbufs[b], gsems[b]
            ).wait()
            pltpu.sync_copy(gbufs[b], acc.at[cidx.at[g]], add=True)
        return 0

    lax.fori_loop(0, CHUNKS // NBUF - 1, ebody, 0)
    for b in range(NBUF):
        g0 = CHUNKS - NBUF + b
        pltpu.make_async_copy(
            hs_hbm.at[ridx.at[pl.ds(g0 * K, K)]], gbufs[b], gsems[b]
        ).wait()
        pltpu.sync_copy(gbufs[b], acc.at[cidx.at[g0]], add=True)
        if b == 0:
            pltpu.async_copy(
                hs_hbm.at[ridx.at[pl.ds((g0 + 1) * K, K)]], gbufs[1], gsems[1]
            )
    plsc.subcore_barrier()

    pltpu.sync_copy(
        acc.at[pl.ds(sid * SL_ACC, SL_ACC)],
        out_hbm.at[cid, pl.ds(sid * SL_ACC, SL_ACC)],
    )


BR = 400


def _mm_body(x_ref, w_ref, dega_ref, degb_ref, hs_ref, dinv_ref):
    deg = dega_ref[...] + degb_ref[...] + 1.0
    dinv = lax.rsqrt(deg)
    h = jnp.dot(x_ref[...], w_ref[...], preferred_element_type=jnp.float32)
    hs_ref[...] = h * dinv
    dinv_ref[...] = dinv


def _fin_body(acc_ref, hs_ref, dinv_ref, b_ref, gamma_ref, beta_ref, out_ref):
    pre = (acc_ref[0] + acc_ref[1] + hs_ref[...]) * dinv_ref[...] + b_ref[...]
    mean = jnp.mean(pre, axis=0, keepdims=True)
    var = jnp.mean(pre * pre, axis=0, keepdims=True) - mean * mean
    xhat = (pre - mean) * lax.rsqrt(var + EPS)
    o = gamma_ref[...] * xhat + beta_ref[...]
    out_ref[...] = o * (1.0 / (1.0 + jnp.exp(-o)))


@jax.jit
def _impl(x, edge_index, W, b, gamma, beta):
    row = edge_index[0]
    col = edge_index[1]
    pad = E_PAD - E
    row_p = jnp.concatenate([row, jnp.zeros((pad,), jnp.int32)]).reshape(NW, EP)
    col_p = jnp.concatenate([col, jnp.full((pad,), N, jnp.int32)]).reshape(NW, CHUNKS, K)

    deg2 = _deg_kernel(col_p)
    dega = deg2[0, :N, 0:1]
    degb = deg2[1, :N, 0:1]

    hs, dinv = pl.pallas_call(
        _mm_body,
        grid=(N // BR,),
        in_specs=[
            pl.BlockSpec((BR, D), lambda i: (i, 0)),
            pl.BlockSpec((D, D), lambda i: (0, 0)),
            pl.BlockSpec((BR, 1), lambda i: (i, 0)),
            pl.BlockSpec((BR, 1), lambda i: (i, 0)),
        ],
        out_specs=[
            pl.BlockSpec((BR, D), lambda i: (i, 0)),
            pl.BlockSpec((BR, 1), lambda i: (i, 0)),
        ],
        out_shape=[
            jax.ShapeDtypeStruct((N, D), jnp.float32),
            jax.ShapeDtypeStruct((N, 1), jnp.float32),
        ],
    )(x, W, dega, degb)

    acc2 = _scatter_kernel(hs, row_p, col_p)[:, :N, :]

    out = pl.pallas_call(
        _fin_body,
        out_shape=jax.ShapeDtypeStruct((N, D), jnp.float32),
    )(acc2, hs, dinv, b.reshape(1, D), gamma.reshape(1, D), beta.reshape(1, D))
    return out


def kernel(x, edge_index, W, b, gamma, beta):
    return _impl(x, edge_index, W, b, gamma, beta)


# speedup vs baseline: 15.1405x; 1.0025x over previous
import functools

import jax
import jax.numpy as jnp
from jax import lax
from jax.experimental import pallas as pl
from jax.experimental.pallas import tpu as pltpu
from jax.experimental.pallas import tpu_sc as plsc

N = 10000
D = 128
E = 320000
EPS = 1e-5

NC = 2
NS = 16
NW = NC * NS
K = 96
CHUNKS = 106
NBUF = 2
EP = K * CHUNKS
E_PAD = EP * NW
N_PAD = 10240
SL_DEG = N_PAD // NS
N_ACC = 10112
SL_ACC = N_ACC // NS
ZR = 16

_mesh = plsc.VectorSubcoreMesh(core_axis_name="c", subcore_axis_name="s")


DW = 128


@functools.partial(
    pl.kernel,
    out_type=jax.ShapeDtypeStruct((NC, N_PAD, DW), jnp.float32),
    mesh=_mesh,
    scratch_types=[
        pltpu.VMEM((CHUNKS, K), jnp.int32),
        pltpu.VMEM((K, DW), jnp.float32),
        pltpu.VMEM((64, DW), jnp.float32),
        pltpu.VMEM_SHARED((N_PAD, DW), jnp.float32),
        pltpu.SemaphoreType.DMA,
    ],
)
def _deg_kernel(col_hbm, deg_out, cidx, onesbuf, zbuf, acc, ssem):
    cid = lax.axis_index("c")
    sid = lax.axis_index("s")
    wid = cid * NS + sid
    zero16 = jnp.zeros((16,), jnp.float32)
    ones16 = jnp.ones((16,), jnp.float32)
    for r in range(K):
        for j in range(DW // 16):
            onesbuf[r, pl.ds(j * 16, 16)] = ones16
    for r in range(64):
        for j in range(DW // 16):
            zbuf[r, pl.ds(j * 16, 16)] = zero16

    pltpu.sync_copy(col_hbm.at[wid], cidx)

    def zibody(i, _):
        pltpu.sync_copy(zbuf, acc.at[pl.ds(sid * SL_DEG + i * 64, 64)])
        return 0

    lax.fori_loop(0, SL_DEG // 64, zibody, 0)
    plsc.subcore_barrier()

    def ebody(g, _):
        pltpu.async_copy(onesbuf, acc.at[cidx.at[g]], ssem, add=True)
        return 0

    lax.fori_loop(0, CHUNKS, ebody, 0)

    def dbody(g, _):
        pltpu.make_async_copy(onesbuf, acc.at[cidx.at[g]], ssem).wait()
        return 0

    lax.fori_loop(0, CHUNKS, dbody, 0)
    plsc.subcore_barrier()

    pltpu.sync_copy(
        acc.at[pl.ds(sid * SL_DEG, SL_DEG)],
        deg_out.at[cid, pl.ds(sid * SL_DEG, SL_DEG)],
    )


@functools.partial(
    pl.kernel,
    out_type=jax.ShapeDtypeStruct((NC, N_ACC, D), jnp.float32),
    mesh=_mesh,
    scratch_types=[
        pltpu.VMEM((EP,), jnp.int32),
        pltpu.VMEM((CHUNKS, K), jnp.int32),
        [pltpu.VMEM((K, D), jnp.float32) for _ in range(NBUF)],
        pltpu.VMEM_SHARED((N_ACC, D), jnp.float32),
        [pltpu.SemaphoreType.DMA for _ in range(NBUF)],
    ],
)
def _scatter_kernel(hs_hbm, row_hbm, col_hbm, out_hbm, ridx, cidx, gbufs, acc, gsems):
    cid = lax.axis_index("c")
    sid = lax.axis_index("s")
    wid = cid * NS + sid
    zero16 = jnp.zeros((16,), jnp.float32)
    for r in range(K):
        for j in range(D // 16):
            gbufs[0][r, pl.ds(j * 16, 16)] = zero16

    pltpu.sync_copy(row_hbm.at[wid], ridx)
    pltpu.sync_copy(col_hbm.at[wid], cidx)

    zbase = sid * SL_ACC
    for i in range(SL_ACC // K):
        pltpu.sync_copy(gbufs[0], acc.at[pl.ds(zbase + i * K, K)])
    zrem = SL_ACC % K
    if zrem:
        pltpu.sync_copy(
            gbufs[0].at[pl.ds(0, zrem)],
            acc.at[pl.ds(zbase + (SL_ACC // K) * K, zrem)],
        )
    plsc.subcore_barrier()

    pltpu.async_copy(hs_hbm.at[ridx.at[pl.ds(0, K)]], gbufs[0], gsems[0])

    def ebody(o, _):
        for b in range(NBUF):
            g = o * NBUF + b
            nb = 1 - b
            pltpu.async_copy(
                hs_hbm.at[ridx.at[pl.ds((g + 1) * K, K)]], gbufs[nb], gsems[nb]
            )
            pltpu.make_async_copy(
                hs_hbm.at[ridx.at[pl.ds(g * K, K)]], gbufs[b], gsems[b]
            ).wait()
            pltpu.sync_copy(gbufs[b], acc.at[cidx.at[g]], add=True)
        return 0

    lax.fori_loop(0, CHUNKS // NBUF - 1, ebody, 0)
    for b in range(NBUF):
        g0 = CHUNKS - NBUF + b
        pltpu.make_async_copy(
            hs_hbm.at[ridx.at[pl.ds(g0 * K, K)]], gbufs[b], gsems[b]
        ).wait()
        pltpu.sync_copy(gbufs[b], acc.at[cidx.at[g0]], add=True)
        if b == 0:
            pltpu.async_copy(
                hs_hbm.at[ridx.at[pl.ds((g0 + 1) * K, K)]], gbufs[1], gsems[1]
            )
    plsc.subcore_barrier()

    pltpu.sync_copy(
        acc.at[pl.ds(sid * SL_ACC, SL_ACC)],
        out_hbm.at[cid, pl.ds(sid * SL_ACC, SL_ACC)],
    )


BR = 400


def _mm_body(x_ref, w_ref, dega_ref, degb_ref, hs_ref, dinv_ref):
    deg = dega_ref[...] + degb_ref[...] + 1.0
    dinv = lax.rsqrt(deg)
    h = jnp.dot(x_ref[...], w_ref[...], preferred_element_type=jnp.float32)
    hs_ref[...] = h * dinv
    dinv_ref[...] = dinv


def _fin_body(acc_ref, hs_ref, dinv_ref, b_ref, gamma_ref, beta_ref, out_ref):
    pre = (acc_ref[0] + acc_ref[1] + hs_ref[...]) * dinv_ref[...] + b_ref[...]
    mean = jnp.mean(pre, axis=0, keepdims=True)
    var = jnp.mean(pre * pre, axis=0, keepdims=True) - mean * mean
    xhat = (pre - mean) * lax.rsqrt(var + EPS)
    o = gamma_ref[...] * xhat + beta_ref[...]
    out_ref[...] = o * (1.0 / (1.0 + jnp.exp(-o)))


@jax.jit
def _impl(x, edge_index, W, b, gamma, beta):
    row = edge_index[0]
    col = edge_index[1]
    pad = E_PAD - E
    row_p = jnp.concatenate([row, jnp.zeros((pad,), jnp.int32)]).reshape(NW, EP)
    dump = N + (jnp.arange(pad, dtype=jnp.int32) % (N_ACC - N))
    col_p = jnp.concatenate([col, dump]).reshape(NW, CHUNKS, K)

    deg2 = _deg_kernel(col_p)
    dega = deg2[0, :N, 0:1]
    degb = deg2[1, :N, 0:1]

    hs, dinv = pl.pallas_call(
        _mm_body,
        grid=(N // BR,),
        in_specs=[
            pl.BlockSpec((BR, D), lambda i: (i, 0)),
            pl.BlockSpec((D, D), lambda i: (0, 0)),
            pl.BlockSpec((BR, 1), lambda i: (i, 0)),
            pl.BlockSpec((BR, 1), lambda i: (i, 0)),
        ],
        out_specs=[
            pl.BlockSpec((BR, D), lambda i: (i, 0)),
            pl.BlockSpec((BR, 1), lambda i: (i, 0)),
        ],
        out_shape=[
            jax.ShapeDtypeStruct((N, D), jnp.float32),
            jax.ShapeDtypeStruct((N, 1), jnp.float32),
        ],
    )(x, W, dega, degb)

    acc2 = _scatter_kernel(hs, row_p, col_p)[:, :N, :]

    out = pl.pallas_call(
        _fin_body,
        out_shape=jax.ShapeDtypeStruct((N, D), jnp.float32),
    )(acc2, hs, dinv, b.reshape(1, D), gamma.reshape(1, D), beta.reshape(1, D))
    return out


def kernel(x, edge_index, W, b, gamma, beta):
    return _impl(x, edge_index, W, b, gamma, beta)
